# baseline (speedup 1.0000x reference)
import functools

import jax
import jax.numpy as jnp
from jax import lax
from jax.experimental import pallas as pl
from jax.experimental.pallas import tpu as pltpu
from jax.experimental.pallas import tpu_sc as plsc

N = 10000
E = 320000
D = 128
K = 10
ALPHA = 0.1

NC = 2
NS = 16
HALF = N // NC
RPT = 320
AGG_ROWS = NS * RPT
GARBAGE = HALF
U = 128
SU = 8 * U
NSU = 315
E_PAD = NSU * SU
RCH = 64
ROW_OFFS = (0, 64, 128, 192, 256)

_mesh = plsc.VectorSubcoreMesh(core_axis_name="c", subcore_axis_name="s")


def _row_base(s):
    return jnp.minimum(s * RPT, HALF - RPT)


@functools.partial(
    pl.kernel,
    out_type=jax.ShapeDtypeStruct((N, 16), jnp.float32),
    mesh=_mesh,
    scratch_types=[
        pltpu.VMEM((32, 16), jnp.int32),
        pltpu.VMEM((U, 16), jnp.float32),
        pltpu.VMEM((1, 16, U), jnp.int32),
        pltpu.VMEM((RCH, 16), jnp.float32),
        pltpu.VMEM_SHARED((AGG_ROWS, 16), jnp.float32),
    ],
)
def _deg_kernel(idx_hbm, meta_hbm, deg_hbm, meta_v, ones_v, idx_v, chunk_v,
                deg_sh):
    c = lax.axis_index("c")
    s = lax.axis_index("s")
    w = c * NS + s
    pltpu.sync_copy(meta_hbm, meta_v)

    def fill(r, _):
        ones_v[r, :] = jnp.full((16,), 1.0, jnp.float32)
        return 0

    lax.fori_loop(0, U, fill, 0)

    def zfill(r, _):
        chunk_v[r, :] = jnp.zeros((16,), jnp.float32)
        return 0

    lax.fori_loop(0, RCH, zfill, 0)

    for off in ROW_OFFS:
        pltpu.sync_copy(chunk_v, deg_sh.at[pl.ds(s * RPT + off, RCH)])
    plsc.subcore_barrier()

    mrow = meta_v[w, :]
    u0 = mrow[0]
    nu = mrow[1]

    def edge_su(i, _):
        pltpu.sync_copy(idx_hbm.at[pl.ds(u0 + i, 1)], idx_v)
        for k in range(8):
            pltpu.sync_copy(ones_v, deg_sh.at[idx_v.at[0, 8 + k]], add=True)
        return 0

    lax.fori_loop(0, nu, edge_su, 0)
    plsc.subcore_barrier()

    lr = _row_base(s)
    for off in ROW_OFFS:
        pltpu.sync_copy(deg_sh.at[pl.ds(lr + off, RCH)], chunk_v)
        pltpu.sync_copy(chunk_v, deg_hbm.at[pl.ds(c * HALF + lr + off, RCH)])


@functools.partial(
    pl.kernel,
    out_type=jax.ShapeDtypeStruct((N, D), jnp.float32),
    mesh=_mesh,
    scratch_types=[
        pltpu.VMEM((32, 16), jnp.int32),
        pltpu.VMEM((1, 16, U), jnp.int32),
        pltpu.VMEM((U, D), jnp.float32),
        pltpu.VMEM((U, D), jnp.float32),
        pltpu.VMEM((RCH, D), jnp.float32),
        pltpu.VMEM((RCH, D), jnp.float32),
        pltpu.VMEM((RCH, D), jnp.float32),
        pltpu.VMEM((RCH, D), jnp.float32),
        pltpu.VMEM((RCH, 16), jnp.float32),
        pltpu.VMEM((RCH, D), jnp.float32),
        pltpu.VMEM_SHARED((AGG_ROWS, D), jnp.float32),
        pltpu.SemaphoreType.DMA,
        pltpu.SemaphoreType.DMA,
        pltpu.SemaphoreType.DMA,
        pltpu.SemaphoreType.DMA,
    ],
)
def _prop_kernel(g_hbm, idx_hbm, a_hbm, c_hbm, meta_hbm, gout_hbm,
                 meta_v, idx_v, gbufa_v, gbufb_v,
                 zero_v, agg_v, gold_v, cvec_v, avec_v, gnew_v, agg_sh,
                 sga, sgb, ssa, ssb):
    c = lax.axis_index("c")
    s = lax.axis_index("s")
    w = c * NS + s
    pltpu.sync_copy(meta_hbm, meta_v)

    def zfill(r, _):
        for k in range(D // 16):
            zero_v[r, pl.ds(k * 16, 16)] = jnp.zeros((16,), jnp.float32)
        return 0

    lax.fori_loop(0, RCH, zfill, 0)
    for off in ROW_OFFS:
        pltpu.sync_copy(zero_v, agg_sh.at[pl.ds(s * RPT + off, RCH)])
    plsc.subcore_barrier()

    mrow = meta_v[w, :]
    u0 = mrow[0]
    nu = mrow[1]

    def edge_su(i, _):
        pltpu.sync_copy(idx_hbm.at[pl.ds(u0 + i, 1)], idx_v)
        pending = pltpu.async_copy(g_hbm.at[idx_v.at[0, 0]], gbufa_v, sga)
        for k in range(8):
            cur_buf = gbufa_v if k % 2 == 0 else gbufb_v
            nxt = None
            if k < 7:
                nxt = pltpu.async_copy(
                    g_hbm.at[idx_v.at[0, k + 1]],
                    gbufb_v if k % 2 == 0 else gbufa_v,
                    sgb if k % 2 == 0 else sga)
            pending.wait()
            pltpu.sync_copy(cur_buf, agg_sh.at[idx_v.at[0, 8 + k]], add=True)
            pending = nxt
        return 0

    lax.fori_loop(0, nu, edge_su, 0)
    plsc.subcore_barrier()

    lr = _row_base(s)
    for off in ROW_OFFS:
        gr = c * HALF + lr + off
        pltpu.sync_copy(agg_sh.at[pl.ds(lr + off, RCH)], agg_v)
        pltpu.sync_copy(g_hbm.at[pl.ds(gr, RCH)], gold_v)
        pltpu.sync_copy(c_hbm.at[pl.ds(gr, RCH)], cvec_v)
        pltpu.sync_copy(a_hbm.at[pl.ds(gr, RCH)], avec_v)

        def rowfn(r, _):
            a_s = avec_v[r, :][0]
            for k in range(D // 16):
                sl = pl.ds(k * 16, 16)
                gnew_v[r, sl] = a_s * (agg_v[r, sl] + gold_v[r, sl]) \
                    + cvec_v[r, sl]
            return 0

        lax.fori_loop(0, RCH, rowfn, 0)
        pltpu.sync_copy(gnew_v, gout_hbm.at[pl.ds(gr, RCH)])


BLK = 1000


def _mlp_body(x_ref, deg_ref, w1_ref, b1_ref, w2_ref, b2_ref,
              g0_ref, a_ref, c_ref, dinv_ref):
    x = x_ref[...]
    h = jnp.dot(x, w1_ref[...].T, preferred_element_type=jnp.float32)
    h = jnp.maximum(h + b1_ref[...], 0.0)
    h = jnp.dot(h, w2_ref[...].T, preferred_element_type=jnp.float32)
    h = h + b2_ref[...]
    deg = deg_ref[...][:, 0:1] + 1.0
    dinv = lax.rsqrt(deg)
    g0 = h * dinv
    g0_ref[...] = g0
    a_ref[...] = jnp.broadcast_to((1.0 - ALPHA) * dinv * dinv, (BLK, 16))
    c_ref[...] = ALPHA * g0
    dinv_ref[...] = dinv


def _mlp_stage(x, deg16, W1, b1, W2, b2):
    grid = (N // BLK,)
    return pl.pallas_call(
        _mlp_body,
        grid=grid,
        in_specs=[
            pl.BlockSpec((BLK, D), lambda i: (i, 0)),
            pl.BlockSpec((BLK, 16), lambda i: (i, 0)),
            pl.BlockSpec((D, D), lambda i: (0, 0)),
            pl.BlockSpec((1, D), lambda i: (0, 0)),
            pl.BlockSpec((D, D), lambda i: (0, 0)),
            pl.BlockSpec((1, D), lambda i: (0, 0)),
        ],
        out_specs=[
            pl.BlockSpec((BLK, D), lambda i: (i, 0)),
            pl.BlockSpec((BLK, 16), lambda i: (i, 0)),
            pl.BlockSpec((BLK, D), lambda i: (i, 0)),
            pl.BlockSpec((BLK, 1), lambda i: (i, 0)),
        ],
        out_shape=[
            jax.ShapeDtypeStruct((N, D), jnp.float32),
            jax.ShapeDtypeStruct((N, 16), jnp.float32),
            jax.ShapeDtypeStruct((N, D), jnp.float32),
            jax.ShapeDtypeStruct((N, 1), jnp.float32),
        ],
    )(x, deg16, W1, b1.reshape(1, D), W2, b2.reshape(1, D))


def _out_body(g_ref, dinv_ref, o_ref):
    h = g_ref[...] / dinv_ref[...]
    m = jnp.max(h, axis=1, keepdims=True)
    ex = jnp.exp(h - m)
    lse = jnp.log(jnp.sum(ex, axis=1, keepdims=True))
    o_ref[...] = h - m - lse


def _out_stage(g, dinv):
    grid = (N // BLK,)
    return pl.pallas_call(
        _out_body,
        grid=grid,
        in_specs=[
            pl.BlockSpec((BLK, D), lambda i: (i, 0)),
            pl.BlockSpec((BLK, 1), lambda i: (i, 0)),
        ],
        out_specs=pl.BlockSpec((BLK, D), lambda i: (i, 0)),
        out_shape=jax.ShapeDtypeStruct((N, D), jnp.float32),
    )(g, dinv)


def _prep_edges(edge_index):
    row = edge_index[0].astype(jnp.int32)
    col = edge_index[1].astype(jnp.int32)
    in0 = col < HALF
    n0 = jnp.sum(in0.astype(jnp.int32))
    pad0 = (-n0) % SU
    b0p = n0 + pad0
    pos0 = jnp.cumsum(in0.astype(jnp.int32)) - 1
    pos1 = b0p + jnp.cumsum((~in0).astype(jnp.int32)) - 1
    pos = jnp.where(in0, pos0, pos1)
    rows_p = jnp.zeros((E_PAD,), jnp.int32).at[pos].set(row)
    col_local = col - jnp.where(in0, 0, HALF).astype(jnp.int32)
    cols_p = jnp.full((E_PAD,), GARBAGE, jnp.int32).at[pos].set(col_local)
    idx3d = jnp.concatenate(
        [rows_p.reshape(NSU, 8, U), cols_p.reshape(NSU, 8, U)], axis=1)

    t0 = b0p // SU
    t1 = NSU - t0
    sar = jnp.arange(NS + 1, dtype=jnp.int32)
    bnd0 = (sar * t0) // NS
    bnd1 = t0 + (sar * t1) // NS
    starts = jnp.concatenate([bnd0[:-1], bnd1[:-1]])
    nums = jnp.concatenate([bnd0[1:] - bnd0[:-1], bnd1[1:] - bnd1[:-1]])
    meta = jnp.zeros((32, 16), jnp.int32)
    meta = meta.at[:, 0].set(starts).at[:, 1].set(nums)
    return idx3d, meta


def kernel(x, edge_index, W1, b1, W2, b2):
    idx3d, meta = _prep_edges(edge_index)
    deg16 = _deg_kernel(idx3d, meta)
    g, avec, cvec, dinv = _mlp_stage(x, deg16, W1, b1, W2, b2)
    for _ in range(K):
        g = _prop_kernel(g, idx3d, avec, cvec, meta)
    return _out_stage(g, dinv)


# speedup vs baseline: 3.0299x; 3.0299x over previous
import functools

import jax
import jax.numpy as jnp
from jax import lax
from jax.experimental import pallas as pl
from jax.experimental.pallas import tpu as pltpu
from jax.experimental.pallas import tpu_sc as plsc

N = 10000
E = 320000
D = 128
K = 10
ALPHA = 0.1

NC = 2
NS = 16
NW = NC * NS
U = 128
SUK = 4
NSU = E // (SUK * U)
ZR = 632
AGG_ROWS = NS * ZR
RCH = 64
ZOFFS = (0, 64, 128, 192, 256, 320, 384, 448, 512, 568)

_mesh = plsc.VectorSubcoreMesh(core_axis_name="c", subcore_axis_name="s")


def _tile_ranges(c, s):
    w = c * NS + s
    u0 = (w * NSU) // NW
    u1 = ((w + 1) * NSU) // NW
    return u0, u1 - u0


def _out_rowbase(s):
    return jnp.minimum(s * ZR, N - ZR)


@functools.partial(
    pl.kernel,
    out_type=jax.ShapeDtypeStruct((NC, N, 16), jnp.float32),
    mesh=_mesh,
    scratch_types=[
        pltpu.VMEM((1, 2 * SUK, U), jnp.int32),
        pltpu.VMEM((U, 16), jnp.float32),
        pltpu.VMEM((RCH, 16), jnp.float32),
        pltpu.VMEM_SHARED((AGG_ROWS, 16), jnp.float32),
    ],
)
def _deg_kernel(idx_hbm, deg_hbm, idx_v, ones_v, zero_v, deg_sh):
    c = lax.axis_index("c")
    s = lax.axis_index("s")
    u0, nu = _tile_ranges(c, s)

    def fill(r, _):
        ones_v[r, :] = jnp.full((16,), 1.0, jnp.float32)
        return 0

    lax.fori_loop(0, U, fill, 0)

    def zfill(r, _):
        zero_v[r, :] = jnp.zeros((16,), jnp.float32)
        return 0

    lax.fori_loop(0, RCH, zfill, 0)
    for off in ZOFFS:
        pltpu.sync_copy(zero_v, deg_sh.at[pl.ds(s * ZR + off, RCH)])
    plsc.subcore_barrier()

    def edge_su(i, _):
        pltpu.sync_copy(idx_hbm.at[pl.ds(u0 + i, 1)], idx_v)
        for k in range(SUK):
            pltpu.sync_copy(ones_v, deg_sh.at[idx_v.at[0, SUK + k]],
                            add=True)
        return 0

    lax.fori_loop(0, nu, edge_su, 0)
    plsc.subcore_barrier()

    lr = _out_rowbase(s)
    pltpu.sync_copy(deg_sh.at[pl.ds(lr, ZR)], deg_hbm.at[c, pl.ds(lr, ZR)])


@functools.partial(
    pl.kernel,
    out_type=jax.ShapeDtypeStruct((NC, N, D), jnp.float32),
    mesh=_mesh,
    scratch_types=[
        pltpu.VMEM((1, 2 * SUK, U), jnp.int32),
        pltpu.VMEM((U, D), jnp.float32),
        pltpu.VMEM((U, D), jnp.float32),
        pltpu.VMEM((RCH, D), jnp.float32),
        pltpu.VMEM_SHARED((AGG_ROWS, D), jnp.float32),
        pltpu.SemaphoreType.DMA,
        pltpu.SemaphoreType.DMA,
    ],
)
def _scatter_kernel(g_hbm, idx_hbm, pout_hbm, idx_v, gbufa_v, gbufb_v,
                    zero_v, agg_sh, sga, sgb):
    c = lax.axis_index("c")
    s = lax.axis_index("s")
    u0, nu = _tile_ranges(c, s)

    def zfill(r, _):
        for k in range(D // 16):
            zero_v[r, pl.ds(k * 16, 16)] = jnp.zeros((16,), jnp.float32)
        return 0

    lax.fori_loop(0, RCH, zfill, 0)
    for off in ZOFFS:
        pltpu.sync_copy(zero_v, agg_sh.at[pl.ds(s * ZR + off, RCH)])
    plsc.subcore_barrier()

    def edge_su(i, _):
        pltpu.sync_copy(idx_hbm.at[pl.ds(u0 + i, 1)], idx_v)
        pending = pltpu.async_copy(g_hbm.at[idx_v.at[0, 0]], gbufa_v, sga)
        for k in range(SUK):
            cur_buf = gbufa_v if k % 2 == 0 else gbufb_v
            nxt = None
            if k < SUK - 1:
                nxt = pltpu.async_copy(
                    g_hbm.at[idx_v.at[0, k + 1]],
                    gbufb_v if k % 2 == 0 else gbufa_v,
                    sgb if k % 2 == 0 else sga)
            pending.wait()
            pltpu.sync_copy(cur_buf, agg_sh.at[idx_v.at[0, SUK + k]],
                            add=True)
            pending = nxt
        return 0

    lax.fori_loop(0, nu, edge_su, 0)
    plsc.subcore_barrier()

    lr = _out_rowbase(s)
    pltpu.sync_copy(agg_sh.at[pl.ds(lr, ZR)], pout_hbm.at[c, pl.ds(lr, ZR)])


BLK = 1000


def _mlp_body(x_ref, deg_ref, w1_ref, b1_ref, w2_ref, b2_ref,
              g0_ref, a_ref, c_ref, dinv_ref):
    x = x_ref[...]
    h = jnp.dot(x, w1_ref[...].T, preferred_element_type=jnp.float32)
    h = jnp.maximum(h + b1_ref[...], 0.0)
    h = jnp.dot(h, w2_ref[...].T, preferred_element_type=jnp.float32)
    h = h + b2_ref[...]
    degs = deg_ref[...]
    deg = degs[0, :, 0:1] + degs[1, :, 0:1] + 1.0
    dinv = lax.rsqrt(deg)
    g0 = h * dinv
    g0_ref[...] = g0
    a_ref[...] = jnp.broadcast_to((1.0 - ALPHA) * dinv * dinv, (BLK, 16))
    c_ref[...] = ALPHA * g0
    dinv_ref[...] = dinv


def _mlp_stage(x, deg2, W1, b1, W2, b2):
    grid = (N // BLK,)
    return pl.pallas_call(
        _mlp_body,
        grid=grid,
        in_specs=[
            pl.BlockSpec((BLK, D), lambda i: (i, 0)),
            pl.BlockSpec((NC, BLK, 16), lambda i: (0, i, 0)),
            pl.BlockSpec((D, D), lambda i: (0, 0)),
            pl.BlockSpec((1, D), lambda i: (0, 0)),
            pl.BlockSpec((D, D), lambda i: (0, 0)),
            pl.BlockSpec((1, D), lambda i: (0, 0)),
        ],
        out_specs=[
            pl.BlockSpec((BLK, D), lambda i: (i, 0)),
            pl.BlockSpec((BLK, 16), lambda i: (i, 0)),
            pl.BlockSpec((BLK, D), lambda i: (i, 0)),
            pl.BlockSpec((BLK, 1), lambda i: (i, 0)),
        ],
        out_shape=[
            jax.ShapeDtypeStruct((N, D), jnp.float32),
            jax.ShapeDtypeStruct((N, 16), jnp.float32),
            jax.ShapeDtypeStruct((N, D), jnp.float32),
            jax.ShapeDtypeStruct((N, 1), jnp.float32),
        ],
    )(x, deg2, W1, b1.reshape(1, D), W2, b2.reshape(1, D))


def _combine_body(p_ref, g_ref, a_ref, c_ref, o_ref):
    p = p_ref[...]
    s = p[0] + p[1] + g_ref[...]
    o_ref[...] = a_ref[...][:, 0:1] * s + c_ref[...]


def _combine_stage(pout, g, avec, cvec):
    grid = (N // BLK,)
    return pl.pallas_call(
        _combine_body,
        grid=grid,
        in_specs=[
            pl.BlockSpec((NC, BLK, D), lambda i: (0, i, 0)),
            pl.BlockSpec((BLK, D), lambda i: (i, 0)),
            pl.BlockSpec((BLK, 16), lambda i: (i, 0)),
            pl.BlockSpec((BLK, D), lambda i: (i, 0)),
        ],
        out_specs=pl.BlockSpec((BLK, D), lambda i: (i, 0)),
        out_shape=jax.ShapeDtypeStruct((N, D), jnp.float32),
    )(pout, g, avec, cvec)


def _out_body(g_ref, dinv_ref, o_ref):
    h = g_ref[...] / dinv_ref[...]
    m = jnp.max(h, axis=1, keepdims=True)
    ex = jnp.exp(h - m)
    lse = jnp.log(jnp.sum(ex, axis=1, keepdims=True))
    o_ref[...] = h - m - lse


def _out_stage(g, dinv):
    grid = (N // BLK,)
    return pl.pallas_call(
        _out_body,
        grid=grid,
        in_specs=[
            pl.BlockSpec((BLK, D), lambda i: (i, 0)),
            pl.BlockSpec((BLK, 1), lambda i: (i, 0)),
        ],
        out_specs=pl.BlockSpec((BLK, D), lambda i: (i, 0)),
        out_shape=jax.ShapeDtypeStruct((N, D), jnp.float32),
    )(g, dinv)


def kernel(x, edge_index, W1, b1, W2, b2):
    row = edge_index[0].astype(jnp.int32)
    col = edge_index[1].astype(jnp.int32)
    idx3d = jnp.concatenate(
        [row.reshape(NSU, SUK, U), col.reshape(NSU, SUK, U)], axis=1)
    deg2 = _deg_kernel(idx3d)
    g, avec, cvec, dinv = _mlp_stage(x, deg2, W1, b1, W2, b2)
    for _ in range(K):
        pout = _scatter_kernel(g, idx3d)
        g = _combine_stage(pout, g, avec, cvec)
    return _out_stage(g, dinv)


# speedup vs baseline: 3.2763x; 1.0813x over previous
import functools

import jax
import jax.numpy as jnp
from jax import lax
from jax.experimental import pallas as pl
from jax.experimental.pallas import tpu as pltpu
from jax.experimental.pallas import tpu_sc as plsc

N = 10000
E = 320000
D = 128
K = 10
ALPHA = 0.1

NC = 2
NS = 16
NW = NC * NS
U = 128
SUK = 4
NSU = E // (SUK * U)
ZR = 632
AGG_ROWS = NS * ZR
RCH = 64
ZOFFS = (0, 64, 128, 192, 256, 320, 384, 448, 512, 568)

_mesh = plsc.VectorSubcoreMesh(core_axis_name="c", subcore_axis_name="s")


def _tile_ranges(c, s):
    w = c * NS + s
    u0 = (w * NSU) // NW
    u1 = ((w + 1) * NSU) // NW
    return u0, u1 - u0


def _out_rowbase(s):
    return jnp.minimum(s * ZR, N - ZR)


@functools.partial(
    pl.kernel,
    out_type=jax.ShapeDtypeStruct((NC, N, 16), jnp.float32),
    mesh=_mesh,
    scratch_types=[
        pltpu.VMEM((1, 2 * SUK, U), jnp.int32),
        pltpu.VMEM((U, 16), jnp.float32),
        pltpu.VMEM((RCH, 16), jnp.float32),
        pltpu.VMEM_SHARED((AGG_ROWS, 16), jnp.float32),
    ],
)
def _deg_kernel(idx_hbm, deg_hbm, idx_v, ones_v, zero_v, deg_sh):
    c = lax.axis_index("c")
    s = lax.axis_index("s")
    u0, nu = _tile_ranges(c, s)

    def fill(r, _):
        ones_v[r, :] = jnp.full((16,), 1.0, jnp.float32)
        return 0

    lax.fori_loop(0, U, fill, 0)

    def zfill(r, _):
        zero_v[r, :] = jnp.zeros((16,), jnp.float32)
        return 0

    lax.fori_loop(0, RCH, zfill, 0)
    for off in ZOFFS:
        pltpu.sync_copy(zero_v, deg_sh.at[pl.ds(s * ZR + off, RCH)])
    plsc.subcore_barrier()

    def edge_su(i, _):
        pltpu.sync_copy(idx_hbm.at[pl.ds(u0 + i, 1)], idx_v)
        for k in range(SUK):
            pltpu.sync_copy(ones_v, deg_sh.at[idx_v.at[0, SUK + k]],
                            add=True)
        return 0

    lax.fori_loop(0, nu, edge_su, 0)
    plsc.subcore_barrier()

    lr = _out_rowbase(s)
    pltpu.sync_copy(deg_sh.at[pl.ds(lr, ZR)], deg_hbm.at[c, pl.ds(lr, ZR)])


@functools.partial(
    pl.kernel,
    out_type=jax.ShapeDtypeStruct((NC, N, D), jnp.float32),
    mesh=_mesh,
    scratch_types=[
        pltpu.VMEM((1, 2 * SUK, U), jnp.int32),
        pltpu.VMEM((1, 2 * SUK, U), jnp.int32),
        pltpu.VMEM((U, D), jnp.float32),
        pltpu.VMEM((U, D), jnp.float32),
        pltpu.VMEM((RCH, D), jnp.float32),
        pltpu.VMEM_SHARED((AGG_ROWS, D), jnp.float32),
        pltpu.SemaphoreType.DMA,
        pltpu.SemaphoreType.DMA,
        pltpu.SemaphoreType.DMA,
        pltpu.SemaphoreType.DMA,
        pltpu.SemaphoreType.DMA,
    ],
)
def _scatter_kernel(g_hbm, idx_hbm, pout_hbm, idxa_v, idxb_v, gbufa_v,
                    gbufb_v, zero_v, agg_sh, sga, sgb, sia, sib, sz):
    c = lax.axis_index("c")
    s = lax.axis_index("s")
    u0, nu = _tile_ranges(c, s)

    def zfill(r, _):
        for k in range(D // 16):
            zero_v[r, pl.ds(k * 16, 16)] = jnp.zeros((16,), jnp.float32)
        return 0

    lax.fori_loop(0, RCH, zfill, 0)
    for off in ZOFFS:
        pltpu.sync_copy(zero_v, agg_sh.at[pl.ds(s * ZR + off, RCH)])
    plsc.subcore_barrier()

    def process_su(idx_v):
        pending = pltpu.async_copy(g_hbm.at[idx_v.at[0, 0]], gbufa_v, sga)
        for k in range(SUK):
            cur_buf = gbufa_v if k % 2 == 0 else gbufb_v
            nxt = None
            if k < SUK - 1:
                nxt = pltpu.async_copy(
                    g_hbm.at[idx_v.at[0, k + 1]],
                    gbufb_v if k % 2 == 0 else gbufa_v,
                    sgb if k % 2 == 0 else sga)
            pending.wait()
            pltpu.sync_copy(cur_buf, agg_sh.at[idx_v.at[0, SUK + k]],
                            add=True)
            pending = nxt

    @pl.when(nu > 0)
    def _prime():
        pltpu.async_copy(idx_hbm.at[pl.ds(u0, 1)], idxa_v, sia)

    def pair(j, _):
        a = u0 + 2 * j
        pltpu.make_async_copy(idx_hbm.at[pl.ds(a, 1)], idxa_v, sia).wait()
        pb = pltpu.async_copy(idx_hbm.at[pl.ds(a + 1, 1)], idxb_v, sib)
        process_su(idxa_v)
        pb.wait()

        @pl.when(2 * j + 2 < nu)
        def _prefetch_next():
            pltpu.async_copy(idx_hbm.at[pl.ds(a + 2, 1)], idxa_v, sia)

        process_su(idxb_v)
        return 0

    lax.fori_loop(0, nu // 2, pair, 0)

    @pl.when(nu % 2 == 1)
    def _tail():
        u = u0 + nu - 1
        pltpu.make_async_copy(idx_hbm.at[pl.ds(u, 1)], idxa_v, sia).wait()
        process_su(idxa_v)

    plsc.subcore_barrier()

    lr = _out_rowbase(s)
    pltpu.sync_copy(agg_sh.at[pl.ds(lr, ZR)], pout_hbm.at[c, pl.ds(lr, ZR)])


BLK = 1000


def _mlp_body(x_ref, deg_ref, w1_ref, b1_ref, w2_ref, b2_ref,
              g0_ref, a_ref, c_ref, dinv_ref):
    x = x_ref[...]
    h = jnp.dot(x, w1_ref[...].T, preferred_element_type=jnp.float32)
    h = jnp.maximum(h + b1_ref[...], 0.0)
    h = jnp.dot(h, w2_ref[...].T, preferred_element_type=jnp.float32)
    h = h + b2_ref[...]
    degs = deg_ref[...]
    deg = degs[0, :, 0:1] + degs[1, :, 0:1] + 1.0
    dinv = lax.rsqrt(deg)
    g0 = h * dinv
    g0_ref[...] = g0
    a_ref[...] = jnp.broadcast_to((1.0 - ALPHA) * dinv * dinv, (BLK, 16))
    c_ref[...] = ALPHA * g0
    dinv_ref[...] = dinv


def _mlp_stage(x, deg2, W1, b1, W2, b2):
    grid = (N // BLK,)
    return pl.pallas_call(
        _mlp_body,
        grid=grid,
        in_specs=[
            pl.BlockSpec((BLK, D), lambda i: (i, 0)),
            pl.BlockSpec((NC, BLK, 16), lambda i: (0, i, 0)),
            pl.BlockSpec((D, D), lambda i: (0, 0)),
            pl.BlockSpec((1, D), lambda i: (0, 0)),
            pl.BlockSpec((D, D), lambda i: (0, 0)),
            pl.BlockSpec((1, D), lambda i: (0, 0)),
        ],
        out_specs=[
            pl.BlockSpec((BLK, D), lambda i: (i, 0)),
            pl.BlockSpec((BLK, 16), lambda i: (i, 0)),
            pl.BlockSpec((BLK, D), lambda i: (i, 0)),
            pl.BlockSpec((BLK, 1), lambda i: (i, 0)),
        ],
        out_shape=[
            jax.ShapeDtypeStruct((N, D), jnp.float32),
            jax.ShapeDtypeStruct((N, 16), jnp.float32),
            jax.ShapeDtypeStruct((N, D), jnp.float32),
            jax.ShapeDtypeStruct((N, 1), jnp.float32),
        ],
    )(x, deg2, W1, b1.reshape(1, D), W2, b2.reshape(1, D))


def _combine_body(p_ref, g_ref, a_ref, c_ref, o_ref):
    p = p_ref[...]
    s = p[0] + p[1] + g_ref[...]
    o_ref[...] = a_ref[...][:, 0:1] * s + c_ref[...]


def _combine_stage(pout, g, avec, cvec):
    grid = (N // BLK,)
    return pl.pallas_call(
        _combine_body,
        grid=grid,
        in_specs=[
            pl.BlockSpec((NC, BLK, D), lambda i: (0, i, 0)),
            pl.BlockSpec((BLK, D), lambda i: (i, 0)),
            pl.BlockSpec((BLK, 16), lambda i: (i, 0)),
            pl.BlockSpec((BLK, D), lambda i: (i, 0)),
        ],
        out_specs=pl.BlockSpec((BLK, D), lambda i: (i, 0)),
        out_shape=jax.ShapeDtypeStruct((N, D), jnp.float32),
    )(pout, g, avec, cvec)


def _out_body(g_ref, dinv_ref, o_ref):
    h = g_ref[...] / dinv_ref[...]
    m = jnp.max(h, axis=1, keepdims=True)
    ex = jnp.exp(h - m)
    lse = jnp.log(jnp.sum(ex, axis=1, keepdims=True))
    o_ref[...] = h - m - lse


def _out_stage(g, dinv):
    grid = (N // BLK,)
    return pl.pallas_call(
        _out_body,
        grid=grid,
        in_specs=[
            pl.BlockSpec((BLK, D), lambda i: (i, 0)),
            pl.BlockSpec((BLK, 1), lambda i: (i, 0)),
        ],
        out_specs=pl.BlockSpec((BLK, D), lambda i: (i, 0)),
        out_shape=jax.ShapeDtypeStruct((N, D), jnp.float32),
    )(g, dinv)


def kernel(x, edge_index, W1, b1, W2, b2):
    row = edge_index[0].astype(jnp.int32)
    col = edge_index[1].astype(jnp.int32)
    idx3d = jnp.concatenate(
        [row.reshape(NSU, SUK, U), col.reshape(NSU, SUK, U)], axis=1)
    deg2 = _deg_kernel(idx3d)
    g, avec, cvec, dinv = _mlp_stage(x, deg2, W1, b1, W2, b2)
    for _ in range(K):
        pout = _scatter_kernel(g, idx3d)
        g = _combine_stage(pout, g, avec, cvec)
    return _out_stage(g, dinv)


# speedup vs baseline: 3.2835x; 1.0022x over previous
import functools

import jax
import jax.numpy as jnp
from jax import lax
from jax.experimental import pallas as pl
from jax.experimental.pallas import tpu as pltpu
from jax.experimental.pallas import tpu_sc as plsc

N = 10000
E = 320000
D = 128
K = 10
ALPHA = 0.1

NC = 2
NS = 16
NW = NC * NS
U = 128
SUK = 4
NSU = E // (SUK * U)
ZR = 632
AGG_ROWS = NS * ZR
RCH = 64
ZOFFS = (0, 64, 128, 192, 256, 320, 384, 448, 512, 568)

_mesh = plsc.VectorSubcoreMesh(core_axis_name="c", subcore_axis_name="s")


def _tile_ranges(c, s):
    w = c * NS + s
    u0 = (w * NSU) // NW
    u1 = ((w + 1) * NSU) // NW
    return u0, u1 - u0


def _out_rowbase(s):
    return jnp.minimum(s * ZR, N - ZR)


@functools.partial(
    pl.kernel,
    out_type=jax.ShapeDtypeStruct((NC, N, 16), jnp.float32),
    mesh=_mesh,
    scratch_types=[
        pltpu.VMEM((1, 2 * SUK, U), jnp.int32),
        pltpu.VMEM((U, 16), jnp.float32),
        pltpu.VMEM((RCH, 16), jnp.float32),
        pltpu.VMEM_SHARED((AGG_ROWS, 16), jnp.float32),
    ],
)
def _deg_kernel(idx_hbm, deg_hbm, idx_v, ones_v, zero_v, deg_sh):
    c = lax.axis_index("c")
    s = lax.axis_index("s")
    u0, nu = _tile_ranges(c, s)

    def fill(r, _):
        ones_v[r, :] = jnp.full((16,), 1.0, jnp.float32)
        return 0

    lax.fori_loop(0, U, fill, 0)

    def zfill(r, _):
        zero_v[r, :] = jnp.zeros((16,), jnp.float32)
        return 0

    lax.fori_loop(0, RCH, zfill, 0)
    for off in ZOFFS:
        pltpu.sync_copy(zero_v, deg_sh.at[pl.ds(s * ZR + off, RCH)])
    plsc.subcore_barrier()

    def edge_su(i, _):
        pltpu.sync_copy(idx_hbm.at[pl.ds(u0 + i, 1)], idx_v)
        for k in range(SUK):
            pltpu.sync_copy(ones_v, deg_sh.at[idx_v.at[0, SUK + k]],
                            add=True)
        return 0

    lax.fori_loop(0, nu, edge_su, 0)
    plsc.subcore_barrier()

    lr = _out_rowbase(s)
    pltpu.sync_copy(deg_sh.at[pl.ds(lr, ZR)], deg_hbm.at[c, pl.ds(lr, ZR)])


@functools.partial(
    pl.kernel,
    out_type=jax.ShapeDtypeStruct((NC, N, D), jnp.float32),
    mesh=_mesh,
    scratch_types=[
        pltpu.VMEM((1, 2 * SUK, U), jnp.int32),
        pltpu.VMEM((1, 2 * SUK, U), jnp.int32),
        pltpu.VMEM((U, D), jnp.float32),
        pltpu.VMEM((U, D), jnp.float32),
        pltpu.VMEM((RCH, D), jnp.float32),
        pltpu.VMEM_SHARED((AGG_ROWS, D), jnp.float32),
        pltpu.SemaphoreType.DMA,
        pltpu.SemaphoreType.DMA,
        pltpu.SemaphoreType.DMA,
        pltpu.SemaphoreType.DMA,
        pltpu.SemaphoreType.DMA,
    ],
)
def _scatter_kernel(g_hbm, idx_hbm, pout_hbm, idxa_v, idxb_v, gbufa_v,
                    gbufb_v, zero_v, agg_sh, sga, sgb, sia, sib, sz):
    c = lax.axis_index("c")
    s = lax.axis_index("s")
    u0, nu = _tile_ranges(c, s)

    @pl.when(nu > 0)
    def _prime_early():
        pltpu.async_copy(idx_hbm.at[pl.ds(u0, 1)], idxa_v, sia)

    def zfill(r, _):
        for k in range(D // 16):
            zero_v[r, pl.ds(k * 16, 16)] = jnp.zeros((16,), jnp.float32)
        return 0

    lax.fori_loop(0, RCH, zfill, 0)
    for off in ZOFFS:
        pltpu.sync_copy(zero_v, agg_sh.at[pl.ds(s * ZR + off, RCH)])
    plsc.subcore_barrier()

    def process_su(idx_v):
        pending = pltpu.async_copy(g_hbm.at[idx_v.at[0, 0]], gbufa_v, sga)
        for k in range(SUK):
            cur_buf = gbufa_v if k % 2 == 0 else gbufb_v
            nxt = None
            if k < SUK - 1:
                nxt = pltpu.async_copy(
                    g_hbm.at[idx_v.at[0, k + 1]],
                    gbufb_v if k % 2 == 0 else gbufa_v,
                    sgb if k % 2 == 0 else sga)
            pending.wait()
            pltpu.sync_copy(cur_buf, agg_sh.at[idx_v.at[0, SUK + k]],
                            add=True)
            pending = nxt

    def pair(j, _):
        a = u0 + 2 * j
        pltpu.make_async_copy(idx_hbm.at[pl.ds(a, 1)], idxa_v, sia).wait()
        pb = pltpu.async_copy(idx_hbm.at[pl.ds(a + 1, 1)], idxb_v, sib)
        process_su(idxa_v)
        pb.wait()

        @pl.when(2 * j + 2 < nu)
        def _prefetch_next():
            pltpu.async_copy(idx_hbm.at[pl.ds(a + 2, 1)], idxa_v, sia)

        process_su(idxb_v)
        return 0

    lax.fori_loop(0, nu // 2, pair, 0)

    @pl.when(nu % 2 == 1)
    def _tail():
        u = u0 + nu - 1
        pltpu.make_async_copy(idx_hbm.at[pl.ds(u, 1)], idxa_v, sia).wait()
        process_su(idxa_v)

    plsc.subcore_barrier()

    lr = _out_rowbase(s)
    pltpu.sync_copy(agg_sh.at[pl.ds(lr, ZR)], pout_hbm.at[c, pl.ds(lr, ZR)])


BLK = 1000


def _mlp_body(x_ref, deg_ref, w1_ref, b1_ref, w2_ref, b2_ref,
              g0_ref, a_ref, c_ref, dinv_ref):
    x = x_ref[...]
    h = jnp.dot(x, w1_ref[...].T, preferred_element_type=jnp.float32)
    h = jnp.maximum(h + b1_ref[...], 0.0)
    h = jnp.dot(h, w2_ref[...].T, preferred_element_type=jnp.float32)
    h = h + b2_ref[...]
    degs = deg_ref[...]
    deg = degs[0, :, 0:1] + degs[1, :, 0:1] + 1.0
    dinv = lax.rsqrt(deg)
    g0 = h * dinv
    g0_ref[...] = g0
    a_ref[...] = jnp.broadcast_to((1.0 - ALPHA) * dinv * dinv, (BLK, 16))
    c_ref[...] = ALPHA * g0
    dinv_ref[...] = dinv


def _mlp_stage(x, deg2, W1, b1, W2, b2):
    grid = (N // BLK,)
    return pl.pallas_call(
        _mlp_body,
        grid=grid,
        in_specs=[
            pl.BlockSpec((BLK, D), lambda i: (i, 0)),
            pl.BlockSpec((NC, BLK, 16), lambda i: (0, i, 0)),
            pl.BlockSpec((D, D), lambda i: (0, 0)),
            pl.BlockSpec((1, D), lambda i: (0, 0)),
            pl.BlockSpec((D, D), lambda i: (0, 0)),
            pl.BlockSpec((1, D), lambda i: (0, 0)),
        ],
        out_specs=[
            pl.BlockSpec((BLK, D), lambda i: (i, 0)),
            pl.BlockSpec((BLK, 16), lambda i: (i, 0)),
            pl.BlockSpec((BLK, D), lambda i: (i, 0)),
            pl.BlockSpec((BLK, 1), lambda i: (i, 0)),
        ],
        out_shape=[
            jax.ShapeDtypeStruct((N, D), jnp.float32),
            jax.ShapeDtypeStruct((N, 16), jnp.float32),
            jax.ShapeDtypeStruct((N, D), jnp.float32),
            jax.ShapeDtypeStruct((N, 1), jnp.float32),
        ],
    )(x, deg2, W1, b1.reshape(1, D), W2, b2.reshape(1, D))


def _combine_body(p_ref, g_ref, a_ref, c_ref, o_ref):
    p = p_ref[...]
    s = p[0] + p[1] + g_ref[...]
    o_ref[...] = a_ref[...][:, 0:1] * s + c_ref[...]


def _combine_stage(pout, g, avec, cvec):
    grid = (N // BLK,)
    return pl.pallas_call(
        _combine_body,
        grid=grid,
        in_specs=[
            pl.BlockSpec((NC, BLK, D), lambda i: (0, i, 0)),
            pl.BlockSpec((BLK, D), lambda i: (i, 0)),
            pl.BlockSpec((BLK, 16), lambda i: (i, 0)),
            pl.BlockSpec((BLK, D), lambda i: (i, 0)),
        ],
        out_specs=pl.BlockSpec((BLK, D), lambda i: (i, 0)),
        out_shape=jax.ShapeDtypeStruct((N, D), jnp.float32),
    )(pout, g, avec, cvec)


def _out_body(g_ref, dinv_ref, o_ref):
    h = g_ref[...] / dinv_ref[...]
    m = jnp.max(h, axis=1, keepdims=True)
    ex = jnp.exp(h - m)
    lse = jnp.log(jnp.sum(ex, axis=1, keepdims=True))
    o_ref[...] = h - m - lse


def _out_stage(g, dinv):
    grid = (N // BLK,)
    return pl.pallas_call(
        _out_body,
        grid=grid,
        in_specs=[
            pl.BlockSpec((BLK, D), lambda i: (i, 0)),
            pl.BlockSpec((BLK, 1), lambda i: (i, 0)),
        ],
        out_specs=pl.BlockSpec((BLK, D), lambda i: (i, 0)),
        out_shape=jax.ShapeDtypeStruct((N, D), jnp.float32),
    )(g, dinv)


def kernel(x, edge_index, W1, b1, W2, b2):
    row = edge_index[0].astype(jnp.int32)
    col = edge_index[1].astype(jnp.int32)
    idx3d = jnp.concatenate(
        [row.reshape(NSU, SUK, U), col.reshape(NSU, SUK, U)], axis=1)
    deg2 = _deg_kernel(idx3d)
    g, avec, cvec, dinv = _mlp_stage(x, deg2, W1, b1, W2, b2)
    for _ in range(K):
        pout = _scatter_kernel(g, idx3d)
        g = _combine_stage(pout, g, avec, cvec)
    return _out_stage(g, dinv)
